# Initial kernel scaffold; baseline (speedup 1.0000x reference)
import jax
import jax.numpy as jnp
from jax.experimental import pallas as pl


def kernel(timestamp, hour_emb, hw_w, hw_b, dd_w, dd_b):
    raise NotImplementedError("write your pallas kernel here")


# speedup vs baseline: 17.3250x; 17.3250x over previous
import functools

import jax
import jax.numpy as jnp
from jax import lax
from jax.experimental import pallas as pl
from jax.experimental.pallas import tpu as pltpu
from jax.experimental.pallas import tpu_sc as plsc

B, L = 4096, 50
LP = 64
NC, NS, LANES = 2, 16, 16
NW = NC * NS
BPW = B // NW
GRP = 16
NGRP = BPW // GRP
ROW = L * L
GBUF = GRP * ROW + LANES


def _div_fix(x, div, inv):
    q = (x.astype(jnp.float32) * inv).astype(jnp.int32)
    r = x - q * div
    neg = r < 0
    q = jnp.where(neg, q - 1, q)
    r = jnp.where(neg, r + div, r)
    ovr = r >= div
    q = jnp.where(ovr, q + 1, q)
    r = jnp.where(ovr, r - div, r)
    return q, r


def _sc_body(ts_hbm, emb_hbm, w_hbm, z_hbm, out_hbm,
             ts_v, emb_v, tab_v, w_v, z_v,
             row_t, row_s, row_hm, row_r, row_qf,
             buf_a, buf_b, sem_a, sem_b):
    wid = lax.axis_index("s") * NC + lax.axis_index("c")
    base = wid * BPW

    pltpu.sync_copy(ts_hbm.at[pl.ds(base, BPW)], ts_v)
    pltpu.sync_copy(emb_hbm, emb_v)
    pltpu.sync_copy(w_hbm, w_v)
    pltpu.sync_copy(z_hbm, z_v)

    for kv in range(5):
        k = lax.iota(jnp.int32, 16) + (16 * kv)
        tab_v[pl.ds(16 * kv, 16)] = plsc.load_gather(emb_v, [lax.rem(k, 24)])

    hw = w_v[0, pl.ds(0, 16)]
    hwb = w_v[1, pl.ds(0, 16)]
    dd = w_v[2, pl.ds(0, 16)]
    ddb = w_v[3, pl.ds(0, 16)]
    zvec = z_v[pl.ds(0, 16)]

    def run_group(g, buf, sem):
        def batch_body(bj, _):
            b = g * GRP + bj
            tj_l, s_l, hm_l, r_l, qf_l = [], [], [], [], []
            for c in range(4):
                t_c = ts_v[b, pl.ds(16 * c, 16)]
                q, r = _div_fix(t_c, 86400, 1.0 / 86400.0)
                hm, s = _div_fix(r, 3600, 1.0 / 3600.0)
                qf = q.astype(jnp.float32)
                row_t[pl.ds(16 * c, 16)] = t_c
                row_s[pl.ds(16 * c, 16)] = s
                row_hm[pl.ds(16 * c, 16)] = hm
                row_r[pl.ds(16 * c, 16)] = r
                row_qf[pl.ds(16 * c, 16)] = qf
                tj_l.append(t_c)
                s_l.append(s)
                hm_l.append(hm)
                r_l.append(r)
                qf_l.append(qf)

            obase = bj * ROW

            def i_body(i, idxv):
                ti = plsc.load_gather(row_t, [idxv])
                si = plsc.load_gather(row_s, [idxv])
                hmi = plsc.load_gather(row_hm, [idxv])
                ri = plsc.load_gather(row_r, [idxv])
                qfi = plsc.load_gather(row_qf, [idxv])
                ai = 24 - hmi
                for c in range(4):
                    m2 = tj_l[c] < ti
                    m1 = s_l[c] < si
                    corr = m2 & (s_l[c] != si)
                    x = hm_l[c] + ai
                    x = jnp.where(m1, x - 1, x)
                    x = jnp.where(corr, x + 1, x)
                    val = plsc.load_gather(tab_v, [x])
                    dday = qf_l[c] - qfi
                    dayf = jnp.where(r_l[c] < ri, dday - 1.0, dday)
                    res = (dayf * hw + hwb) * val + (dayf * dd + ddb)
                    buf[pl.ds(obase + i * L + 16 * c, 16)] = res
                return idxv + 1

            lax.fori_loop(0, L, i_body, zvec)
            return 0

        lax.fori_loop(0, GRP, batch_body, 0)
        pltpu.make_async_copy(
            buf.at[pl.ds(0, GRP * ROW)],
            out_hbm.at[pl.ds((base + g * GRP) * ROW, GRP * ROW)],
            sem,
        ).start()

    def wait_group(g, buf, sem):
        pltpu.make_async_copy(
            buf.at[pl.ds(0, GRP * ROW)],
            out_hbm.at[pl.ds((base + g * GRP) * ROW, GRP * ROW)],
            sem,
        ).wait()

    for g in range(NGRP):
        buf, sem = (buf_a, sem_a) if g % 2 == 0 else (buf_b, sem_b)
        if g >= 2:
            wait_group(g - 2, buf, sem)
        run_group(g, buf, sem)
    wait_group(NGRP - 2, buf_a, sem_a)
    wait_group(NGRP - 1, buf_b, sem_b)


@jax.jit
def kernel(timestamp, hour_emb, hw_w, hw_b, dd_w, dd_b):
    ts_pad = jnp.pad(timestamp, ((0, 0), (0, LP - L)))
    emb = hour_emb.reshape(24)
    wmat = jnp.stack([
        jnp.broadcast_to(hw_w.reshape(1), (16,)),
        jnp.broadcast_to(hw_b.reshape(1), (16,)),
        jnp.broadcast_to(dd_w.reshape(1), (16,)),
        jnp.broadcast_to(dd_b.reshape(1), (16,)),
    ])
    zvec = jnp.zeros((16,), jnp.int32)
    sc = pl.kernel(
        _sc_body,
        out_type=jax.ShapeDtypeStruct((B * ROW,), jnp.float32),
        mesh=plsc.VectorSubcoreMesh(core_axis_name="c", subcore_axis_name="s"),
        compiler_params=pltpu.CompilerParams(needs_layout_passes=False),
        scratch_types=[
            pltpu.VMEM((BPW, LP), jnp.int32),
            pltpu.VMEM((24,), jnp.float32),
            pltpu.VMEM((80,), jnp.float32),
            pltpu.VMEM((4, 16), jnp.float32),
            pltpu.VMEM((16,), jnp.int32),
            pltpu.VMEM((LP,), jnp.int32),
            pltpu.VMEM((LP,), jnp.int32),
            pltpu.VMEM((LP,), jnp.int32),
            pltpu.VMEM((LP,), jnp.int32),
            pltpu.VMEM((LP,), jnp.float32),
            pltpu.VMEM((GBUF,), jnp.float32),
            pltpu.VMEM((GBUF,), jnp.float32),
            pltpu.SemaphoreType.DMA,
            pltpu.SemaphoreType.DMA,
        ],
    )
    out = sc(ts_pad, emb, wmat, zvec)
    return out.reshape(B, L, L)


# speedup vs baseline: 20.2025x; 1.1661x over previous
import functools

import jax
import jax.numpy as jnp
from jax import lax
from jax.experimental import pallas as pl
from jax.experimental.pallas import tpu as pltpu
from jax.experimental.pallas import tpu_sc as plsc

B, L = 4096, 50
LP = 64
NC, NS, LANES = 2, 16, 16
NW = NC * NS
BPW = B // NW
GRP = 16
NGRP = BPW // GRP
ROW = L * L
GBUF = GRP * ROW + LANES


def _div_fix(x, div, inv):
    q = (x.astype(jnp.float32) * inv).astype(jnp.int32)
    r = x - q * div
    neg = r < 0
    q = jnp.where(neg, q - 1, q)
    r = jnp.where(neg, r + div, r)
    ovr = r >= div
    q = jnp.where(ovr, q + 1, q)
    r = jnp.where(ovr, r - div, r)
    return q, r


_GATHER_DNUMS = lax.GatherDimensionNumbers(
    offset_dims=(), collapsed_slice_dims=(0,), start_index_map=(0,))


def _lane_bcast(v, idx):
    return lax.gather(v, idx[:, None], _GATHER_DNUMS, (1,),
                      mode=lax.GatherScatterMode.PROMISE_IN_BOUNDS)


def _sc_body(ts_hbm, emb_hbm, w_hbm, z_hbm, out_hbm,
             ts_v, emb_v, tab_v, w_v, z_v,
             buf_a, buf_b, sem_a, sem_b):
    wid = lax.axis_index("s") * NC + lax.axis_index("c")
    base = wid * BPW

    pltpu.sync_copy(ts_hbm.at[pl.ds(base, BPW)], ts_v)
    pltpu.sync_copy(emb_hbm, emb_v)
    pltpu.sync_copy(w_hbm, w_v)
    pltpu.sync_copy(z_hbm, z_v)

    for kv in range(5):
        k = lax.iota(jnp.int32, 16) + (16 * kv)
        tab_v[pl.ds(16 * kv, 16)] = plsc.load_gather(emb_v, [lax.rem(k, 24)])

    hw = w_v[0, pl.ds(0, 16)]
    hwb = w_v[1, pl.ds(0, 16)]
    dd = w_v[2, pl.ds(0, 16)]
    ddb = w_v[3, pl.ds(0, 16)]
    zvec = z_v[pl.ds(0, 16)]

    def run_group(g, buf, sem):
        def batch_body(bj, _):
            b = g * GRP + bj
            tj_l, s_l, hm_l, r_l, qf_l = [], [], [], [], []
            for c in range(4):
                t_c = ts_v[b, pl.ds(16 * c, 16)]
                q, r = _div_fix(t_c, 86400, 1.0 / 86400.0)
                hm, s = _div_fix(r, 3600, 1.0 / 3600.0)
                qf = q.astype(jnp.float32)
                tj_l.append(t_c)
                s_l.append(s)
                hm_l.append(hm)
                r_l.append(r)
                qf_l.append(qf)

            obase = bj * ROW

            for ic in range(4):
                ni = L - 16 * ic if ic == 3 else 16

                def i_body(k, idxv, ic=ic):
                    ti = _lane_bcast(tj_l[ic], idxv)
                    si = _lane_bcast(s_l[ic], idxv)
                    hmi = _lane_bcast(hm_l[ic], idxv)
                    ri = _lane_bcast(r_l[ic], idxv)
                    qfi = _lane_bcast(qf_l[ic], idxv)
                    ai = 24 - hmi
                    off = obase + (16 * ic) * L + k * L
                    for c in range(4):
                        m2 = tj_l[c] < ti
                        m1 = s_l[c] < si
                        corr = m2 & (s_l[c] != si)
                        x = hm_l[c] + ai
                        x = jnp.where(m1, x - 1, x)
                        x = jnp.where(corr, x + 1, x)
                        val = plsc.load_gather(tab_v, [x])
                        dday = qf_l[c] - qfi
                        dayf = jnp.where(r_l[c] < ri, dday - 1.0, dday)
                        res = (dayf * hw + hwb) * val + (dayf * dd + ddb)
                        buf[pl.ds(off + 16 * c, 16)] = res
                    return idxv + 1

                lax.fori_loop(0, ni, i_body, zvec)
            return 0

        lax.fori_loop(0, GRP, batch_body, 0)
        pltpu.make_async_copy(
            buf.at[pl.ds(0, GRP * ROW)],
            out_hbm.at[pl.ds((base + g * GRP) * ROW, GRP * ROW)],
            sem,
        ).start()

    def wait_group(g, buf, sem):
        pltpu.make_async_copy(
            buf.at[pl.ds(0, GRP * ROW)],
            out_hbm.at[pl.ds((base + g * GRP) * ROW, GRP * ROW)],
            sem,
        ).wait()

    for g in range(NGRP):
        buf, sem = (buf_a, sem_a) if g % 2 == 0 else (buf_b, sem_b)
        if g >= 2:
            wait_group(g - 2, buf, sem)
        run_group(g, buf, sem)
    wait_group(NGRP - 2, buf_a, sem_a)
    wait_group(NGRP - 1, buf_b, sem_b)


@jax.jit
def kernel(timestamp, hour_emb, hw_w, hw_b, dd_w, dd_b):
    ts_pad = jnp.pad(timestamp, ((0, 0), (0, LP - L)))
    emb = hour_emb.reshape(24)
    wmat = jnp.stack([
        jnp.broadcast_to(hw_w.reshape(1), (16,)),
        jnp.broadcast_to(hw_b.reshape(1), (16,)),
        jnp.broadcast_to(dd_w.reshape(1), (16,)),
        jnp.broadcast_to(dd_b.reshape(1), (16,)),
    ])
    zvec = jnp.zeros((16,), jnp.int32)
    sc = pl.kernel(
        _sc_body,
        out_type=jax.ShapeDtypeStruct((B * ROW,), jnp.float32),
        mesh=plsc.VectorSubcoreMesh(core_axis_name="c", subcore_axis_name="s"),
        compiler_params=pltpu.CompilerParams(needs_layout_passes=False),
        scratch_types=[
            pltpu.VMEM((BPW, LP), jnp.int32),
            pltpu.VMEM((24,), jnp.float32),
            pltpu.VMEM((80,), jnp.float32),
            pltpu.VMEM((4, 16), jnp.float32),
            pltpu.VMEM((16,), jnp.int32),
            pltpu.VMEM((GBUF,), jnp.float32),
            pltpu.VMEM((GBUF,), jnp.float32),
            pltpu.SemaphoreType.DMA,
            pltpu.SemaphoreType.DMA,
        ],
    )
    out = sc(ts_pad, emb, wmat, zvec)
    return out.reshape(B, L, L)
